# baseline (speedup 1.0000x reference)
import functools

import jax
import jax.numpy as jnp
from jax import lax
from jax.experimental import pallas as pl
from jax.experimental.pallas import tpu as pltpu
from jax.experimental.pallas import tpu_sc as plsc

N = 10000
E = 320000
D = 128
H = 128
O = 128
G = 64

NC = 2
NS = 16
NW = NC * NS
EPT = E // NW
CH = 128
NCHUNK = 80
PCH = NCHUNK + 1
ZCHUNKS = ((0, 128), (128, 128), (256, 128), (384, 128), (512, 112))

B = 2000
NB = N // B


def _leaky(v):
    return jnp.where(v > 0, v, 0.2 * v)


def _tc_prep_body(x_ref, w1_ref, b1_ref, wg_ref, att_ref,
                  h_ref, hw_ref, asad_ref):
    h = jnp.dot(x_ref[...], w1_ref[...], preferred_element_type=jnp.float32)
    h = h + b1_ref[...]
    hw = jnp.dot(h, wg_ref[...], preferred_element_type=jnp.float32)
    asad = lax.dot_general(hw, att_ref[...], (((1,), (1,)), ((), ())),
                           preferred_element_type=jnp.float32)
    h_ref[...] = h
    hw_ref[...] = hw
    asad_ref[...] = asad


def _tc_prep(x, W1, b1r, Wg0, att0):
    return pl.pallas_call(
        _tc_prep_body,
        grid=(NB,),
        in_specs=[
            pl.BlockSpec((B, D), lambda i: (i, 0)),
            pl.BlockSpec((D, H), lambda i: (0, 0)),
            pl.BlockSpec((1, H), lambda i: (0, 0)),
            pl.BlockSpec((H, H), lambda i: (0, 0)),
            pl.BlockSpec((2, H), lambda i: (0, 0)),
        ],
        out_specs=[
            pl.BlockSpec((B, H), lambda i: (i, 0)),
            pl.BlockSpec((B, H), lambda i: (i, 0)),
            pl.BlockSpec((B, 2), lambda i: (i, 0)),
        ],
        out_shape=[
            jax.ShapeDtypeStruct((N, H), jnp.float32),
            jax.ShapeDtypeStruct((N, H), jnp.float32),
            jax.ShapeDtypeStruct((N, 2), jnp.float32),
        ],
    )(x, W1, b1r, Wg0, att0)


def _epilogue_h(acc2_ref, s32_ref, hw_ref, asad_ref, hprev_ref, bg_ref):
    es = jnp.sum(asad_ref[...], axis=1, keepdims=True)
    ws = jnp.exp(_leaky(es))
    acc = acc2_ref[0] + acc2_ref[1] + ws * hw_ref[...]
    s = jnp.sum(s32_ref[...], axis=1, keepdims=True)
    stot = s + ws
    out = acc / (stot + 1e-16) + bg_ref[...]
    return jnp.maximum(out, 0.0) + hprev_ref[...]


def _tc_mid_body(acc2_ref, s32_ref, hw_ref, asad_ref, hprev_ref, bg_ref,
                 wg_ref, att_ref, h_ref, hwn_ref, asadn_ref):
    h = _epilogue_h(acc2_ref, s32_ref, hw_ref, asad_ref, hprev_ref, bg_ref)
    hw = jnp.dot(h, wg_ref[...], preferred_element_type=jnp.float32)
    asad = lax.dot_general(hw, att_ref[...], (((1,), (1,)), ((), ())),
                           preferred_element_type=jnp.float32)
    h_ref[...] = h
    hwn_ref[...] = hw
    asadn_ref[...] = asad


def _tc_mid(acc2, s32, hw, asad, hprev, bgr, Wg1, att1):
    return pl.pallas_call(
        _tc_mid_body,
        grid=(NB,),
        in_specs=[
            pl.BlockSpec((NC, B, H), lambda i: (0, i, 0)),
            pl.BlockSpec((B, NW), lambda i: (i, 0)),
            pl.BlockSpec((B, H), lambda i: (i, 0)),
            pl.BlockSpec((B, 2), lambda i: (i, 0)),
            pl.BlockSpec((B, H), lambda i: (i, 0)),
            pl.BlockSpec((1, H), lambda i: (0, 0)),
            pl.BlockSpec((H, H), lambda i: (0, 0)),
            pl.BlockSpec((2, H), lambda i: (0, 0)),
        ],
        out_specs=[
            pl.BlockSpec((B, H), lambda i: (i, 0)),
            pl.BlockSpec((B, H), lambda i: (i, 0)),
            pl.BlockSpec((B, 2), lambda i: (i, 0)),
        ],
        out_shape=[
            jax.ShapeDtypeStruct((N, H), jnp.float32),
            jax.ShapeDtypeStruct((N, H), jnp.float32),
            jax.ShapeDtypeStruct((N, 2), jnp.float32),
        ],
    )(acc2, s32, hw, asad, hprev, bgr, Wg1, att1)


def _tc_final_body(acc2_ref, s32_ref, hw_ref, asad_ref, hprev_ref, bg_ref,
                   batch_ref, w2_ref, b2_ref, out_ref, pooled_ref):
    i = pl.program_id(0)
    h = _epilogue_h(acc2_ref, s32_ref, hw_ref, asad_ref, hprev_ref, bg_ref)
    gids = lax.broadcasted_iota(jnp.int32, (1, G), 1)
    onehot = (batch_ref[...] == gids).astype(jnp.float32)
    part = lax.dot_general(onehot, h, (((0,), (0,)), ((), ())),
                           preferred_element_type=jnp.float32)

    @pl.when(i == 0)
    def _():
        pooled_ref[...] = part

    @pl.when(i > 0)
    def _():
        pooled_ref[...] = pooled_ref[...] + part

    @pl.when(i == NB - 1)
    def _():
        out_ref[...] = jnp.dot(pooled_ref[...], w2_ref[...],
                               preferred_element_type=jnp.float32) + b2_ref[...]


def _tc_final(acc2, s32, hw, asad, hprev, bgr, batch_col, W2, b2r):
    return pl.pallas_call(
        _tc_final_body,
        grid=(NB,),
        in_specs=[
            pl.BlockSpec((NC, B, H), lambda i: (0, i, 0)),
            pl.BlockSpec((B, NW), lambda i: (i, 0)),
            pl.BlockSpec((B, H), lambda i: (i, 0)),
            pl.BlockSpec((B, 2), lambda i: (i, 0)),
            pl.BlockSpec((B, H), lambda i: (i, 0)),
            pl.BlockSpec((1, H), lambda i: (0, 0)),
            pl.BlockSpec((B, 1), lambda i: (i, 0)),
            pl.BlockSpec((H, O), lambda i: (0, 0)),
            pl.BlockSpec((1, O), lambda i: (0, 0)),
        ],
        out_specs=pl.BlockSpec((G, O), lambda i: (0, 0)),
        out_shape=jax.ShapeDtypeStruct((G, O), jnp.float32),
        scratch_shapes=[pltpu.VMEM((G, H), jnp.float32)],
    )(acc2, s32, hw, asad, hprev, bgr, batch_col, W2, b2r)


def _sc_edge_pass(hw, a_src, a_dst, packed):
    mesh = plsc.VectorSubcoreMesh(core_axis_name="c", subcore_axis_name="s")

    @functools.partial(
        pl.kernel,
        mesh=mesh,
        compiler_params=pltpu.CompilerParams(needs_layout_passes=False),
        out_type=(
            jax.ShapeDtypeStruct((NC, N, H), jnp.float32),
            jax.ShapeDtypeStruct((NW, 1, N), jnp.float32),
        ),
        scratch_types=[
            pltpu.VMEM((CH, H), jnp.float32),
            pltpu.VMEM((CH, H), jnp.float32),
            pltpu.VMEM((N,), jnp.float32),
            pltpu.VMEM((1, CH), jnp.int32),
            pltpu.VMEM((1, CH), jnp.int32),
            pltpu.VMEM((CH,), jnp.int32),
            pltpu.VMEM((CH,), jnp.int32),
            pltpu.VMEM((CH,), jnp.int32),
            pltpu.VMEM((CH,), jnp.int32),
            pltpu.VMEM((CH,), jnp.float32),
            pltpu.VMEM((CH,), jnp.float32),
            pltpu.VMEM((CH,), jnp.float32),
            pltpu.VMEM((CH,), jnp.float32),
            pltpu.VMEM((CH,), jnp.float32),
            pltpu.VMEM((CH,), jnp.float32),
            pltpu.VMEM_SHARED((N, H), jnp.float32),
            pltpu.SemaphoreType.DMA,
            pltpu.SemaphoreType.DMA,
            pltpu.SemaphoreType.DMA,
            pltpu.SemaphoreType.DMA,
            pltpu.SemaphoreType.DMA,
            pltpu.SemaphoreType.DMA,
        ],
    )
    def k(hw_hbm, as_hbm, ad_hbm, pk_hbm, acc_out, s_out,
          rows0, rows1, s_loc, pkb0, pkb1, sidx0, sidx1, didx0, didx1,
          asb0, asb1, adb0, adb1, wb0, wb1, acc_sh,
          gsem0, gsem1, ssem0, ssem1, psem0, psem1):
        cid = lax.axis_index("c")
        sid = lax.axis_index("s")
        wid = sid * NC + cid
        z16 = jnp.zeros((16,), jnp.float32)

        def zrow(j, _):
            for kk in range(H // 16):
                rows0[j, pl.ds(kk * 16, 16)] = z16
            return 0
        lax.fori_loop(0, CH, zrow, 0)

        rbase = pl.multiple_of(sid * 624 + jnp.minimum(sid, 2) * 8, 8)
        for roff, nr in ZCHUNKS:
            pltpu.sync_copy(rows0.at[pl.ds(0, nr)],
                            acc_sh.at[pl.ds(rbase + roff, nr)])

        @pl.when(sid < 2)
        def _():
            pltpu.sync_copy(rows0.at[pl.ds(0, 8)],
                            acc_sh.at[pl.ds(rbase + 624, 8)])

        def zs(j, _):
            s_loc[pl.ds(j * 16, 16)] = z16
            return 0
        lax.fori_loop(0, N // 16, zs, 0)

        plsc.subcore_barrier()

        lane = lax.iota(jnp.int32, 16)
        m16 = jnp.full((16,), 0xFFFF, jnp.int32)

        def pkload(c, pkb, psem):
            pltpu.async_copy(pk_hbm.at[wid, jnp.minimum(c, PCH - 1)], pkb,
                             psem)

        def pkwait(c, pkb, psem):
            pltpu.make_async_copy(
                pk_hbm.at[wid, jnp.minimum(c, PCH - 1)], pkb, psem).wait()

        def unpack(pkb, sidx, didx):
            for g in range(CH // 16):
                p = pkb[0, pl.ds(g * 16, 16)]
                sidx[pl.ds(g * 16, 16)] = p & m16
                didx[pl.ds(g * 16, 16)] = lax.shift_right_logical(p, 16)

        def issue_gathers(sidx, didx, rows, asb, adb, gsem):
            pltpu.async_copy(hw_hbm.at[sidx], rows, gsem)
            pltpu.async_copy(as_hbm.at[sidx], asb, gsem)
            pltpu.async_copy(ad_hbm.at[didx], adb, gsem)

        def wait_gathers(sidx, didx, rows, asb, adb, gsem):
            pltpu.make_async_copy(hw_hbm.at[sidx], rows, gsem).wait()
            pltpu.make_async_copy(as_hbm.at[sidx], asb, gsem).wait()
            pltpu.make_async_copy(ad_hbm.at[didx], adb, gsem).wait()

        pkload(0, pkb0, psem0)
        pkwait(0, pkb0, psem0)
        unpack(pkb0, sidx0, didx0)
        issue_gathers(sidx0, didx0, rows0, asb0, adb0, gsem0)
        pkload(1, pkb1, psem1)

        def _do_chunk(c, cur, nxt):
            (cur_rows, cur_pkb, cur_sidx, cur_didx, cur_asb, cur_adb, cur_wb,
             cur_gsem, cur_ssem, cur_psem) = cur
            (nxt_rows, nxt_pkb, nxt_sidx, nxt_didx, nxt_asb, nxt_adb, nxt_wb,
             nxt_gsem, nxt_ssem, nxt_psem) = nxt

            wait_gathers(cur_sidx, cur_didx, cur_rows, cur_asb, cur_adb,
                         cur_gsem)

            for g in range(CH // 16):
                a1 = cur_asb[pl.ds(g * 16, 16)]
                a2 = cur_adb[pl.ds(g * 16, 16)]
                w = jnp.exp(_leaky(a1 + a2))
                eoff = c * CH + g * 16 + lane
                w = jnp.where(eoff < EPT, w, 0.0)
                cur_wb[pl.ds(g * 16, 16)] = w
                plsc.addupdate_scatter(s_loc, [cur_didx[pl.ds(g * 16, 16)]],
                                       w)

            @pl.when(c >= 1)
            def _():
                pltpu.make_async_copy(nxt_rows, acc_sh.at[nxt_didx],
                                      nxt_ssem).wait()

            pkwait(c + 1, nxt_pkb, nxt_psem)
            unpack(nxt_pkb, nxt_sidx, nxt_didx)
            issue_gathers(nxt_sidx, nxt_didx, nxt_rows, nxt_asb, nxt_adb,
                          nxt_gsem)
            pkload(c + 2, cur_pkb, cur_psem)

            @plsc.parallel_loop(0, CH, unroll=4)
            def _(j):
                wv = plsc.load_gather(cur_wb, [jnp.full((16,), j, jnp.int32)])
                for kk in range(H // 16):
                    cur_rows[j, pl.ds(kk * 16, 16)] = \
                        cur_rows[j, pl.ds(kk * 16, 16)] * wv

            pltpu.async_copy(cur_rows, acc_sh.at[cur_didx], cur_ssem,
                             add=True)

        buf0 = (rows0, pkb0, sidx0, didx0, asb0, adb0, wb0, gsem0, ssem0,
                psem0)
        buf1 = (rows1, pkb1, sidx1, didx1, asb1, adb1, wb1, gsem1, ssem1,
                psem1)

        def chunk_pair(j, _):
            _do_chunk(2 * j, buf0, buf1)
            _do_chunk(2 * j + 1, buf1, buf0)
            return 0

        lax.fori_loop(0, NCHUNK // 2, chunk_pair, 0)

        pltpu.make_async_copy(rows1, acc_sh.at[didx1], ssem1).wait()
        wait_gathers(sidx0, didx0, rows0, asb0, adb0, gsem0)
        pkwait(NCHUNK + 1, pkb1, psem1)

        plsc.subcore_barrier()

        for roff, nr in ZCHUNKS:
            pltpu.sync_copy(acc_sh.at[pl.ds(rbase + roff, nr)],
                            acc_out.at[cid, pl.ds(rbase + roff, nr)])

        @pl.when(sid < 2)
        def _():
            pltpu.sync_copy(acc_sh.at[pl.ds(rbase + 624, 8)],
                            acc_out.at[cid, pl.ds(rbase + 624, 8)])

        pltpu.sync_copy(s_loc, s_out.at[wid, 0])

    return k(hw, a_src, a_dst, packed)


def _pack_edges(v):
    v2 = v.reshape(NW, EPT)
    v2 = jnp.pad(v2, ((0, 0), (0, PCH * CH - EPT)))
    return v2.reshape(NW, PCH, 1, CH)


def kernel(x, edge_index, batch, W1, b1, Wg, att_src, att_dst, bg, W2, b2):
    packed = _pack_edges(
        jnp.bitwise_or(jnp.left_shift(edge_index[1].astype(jnp.int32), 16),
                       edge_index[0].astype(jnp.int32)))
    batch_col = batch.astype(jnp.int32).reshape(N, 1)
    b1r = b1.reshape(1, H)
    b2r = b2.reshape(1, O)
    att0 = jnp.stack([att_src[0], att_dst[0]])
    att1 = jnp.stack([att_src[1], att_dst[1]])
    bg0 = bg[0].reshape(1, H)
    bg1 = bg[1].reshape(1, H)

    h0, hw0, asad0 = _tc_prep(x, W1, b1r, Wg[0], att0)
    acc0, s0 = _sc_edge_pass(hw0, asad0[:, 0], asad0[:, 1], packed)
    h1, hw1, asad1 = _tc_mid(acc0, s0.reshape(NW, N).T, hw0, asad0, h0,
                             bg0, Wg[1], att1)
    acc1, s1 = _sc_edge_pass(hw1, asad1[:, 0], asad1[:, 1], packed)
    return _tc_final(acc1, s1.reshape(NW, N).T, hw1, asad1, h1, bg1,
                     batch_col, W2, b2r)


# speedup vs baseline: 1.6265x; 1.6265x over previous
import functools

import jax
import jax.numpy as jnp
from jax import lax
from jax.experimental import pallas as pl
from jax.experimental.pallas import tpu as pltpu
from jax.experimental.pallas import tpu_sc as plsc

N = 10000
E = 320000
D = 128
H = 128
O = 128
G = 64

NC = 2
NS = 16
NW = NC * NS
EPT = E // NW
CH = 128
NCHUNK = (EPT + CH - 1) // CH
EPAD = (NW - 1) * EPT + NCHUNK * CH
ZCHUNKS = ((0, 128), (128, 128), (256, 128), (384, 128), (512, 112))

B = 2000
NB = N // B


def _leaky(v):
    return jnp.where(v > 0, v, 0.2 * v)


def _tc_prep_body(x_ref, w1_ref, b1_ref, wg_ref, att_ref,
                  h_ref, hw_ref, asad_ref):
    h = jnp.dot(x_ref[...], w1_ref[...], preferred_element_type=jnp.float32)
    h = h + b1_ref[...]
    hw = jnp.dot(h, wg_ref[...], preferred_element_type=jnp.float32)
    asad = lax.dot_general(hw, att_ref[...], (((1,), (1,)), ((), ())),
                           preferred_element_type=jnp.float32)
    h_ref[...] = h
    hw_ref[...] = hw
    asad_ref[...] = asad


def _tc_prep(x, W1, b1r, Wg0, att0):
    return pl.pallas_call(
        _tc_prep_body,
        grid=(NB,),
        in_specs=[
            pl.BlockSpec((B, D), lambda i: (i, 0)),
            pl.BlockSpec((D, H), lambda i: (0, 0)),
            pl.BlockSpec((1, H), lambda i: (0, 0)),
            pl.BlockSpec((H, H), lambda i: (0, 0)),
            pl.BlockSpec((2, H), lambda i: (0, 0)),
        ],
        out_specs=[
            pl.BlockSpec((B, H), lambda i: (i, 0)),
            pl.BlockSpec((B, H), lambda i: (i, 0)),
            pl.BlockSpec((B, 2), lambda i: (i, 0)),
        ],
        out_shape=[
            jax.ShapeDtypeStruct((N, H), jnp.float32),
            jax.ShapeDtypeStruct((N, H), jnp.float32),
            jax.ShapeDtypeStruct((N, 2), jnp.float32),
        ],
    )(x, W1, b1r, Wg0, att0)


def _epilogue_h(acc2_ref, s32_ref, hw_ref, asad_ref, hprev_ref, bg_ref):
    es = jnp.sum(asad_ref[...], axis=1, keepdims=True)
    ws = jnp.exp(_leaky(es))
    acc = acc2_ref[0] + acc2_ref[1] + ws * hw_ref[...]
    s = jnp.sum(s32_ref[...], axis=1, keepdims=True)
    stot = s + ws
    out = acc / (stot + 1e-16) + bg_ref[...]
    return jnp.maximum(out, 0.0) + hprev_ref[...]


def _tc_mid_body(acc2_ref, s32_ref, hw_ref, asad_ref, hprev_ref, bg_ref,
                 wg_ref, att_ref, h_ref, hwn_ref, asadn_ref):
    h = _epilogue_h(acc2_ref, s32_ref, hw_ref, asad_ref, hprev_ref, bg_ref)
    hw = jnp.dot(h, wg_ref[...], preferred_element_type=jnp.float32)
    asad = lax.dot_general(hw, att_ref[...], (((1,), (1,)), ((), ())),
                           preferred_element_type=jnp.float32)
    h_ref[...] = h
    hwn_ref[...] = hw
    asadn_ref[...] = asad


def _tc_mid(acc2, s32, hw, asad, hprev, bgr, Wg1, att1):
    return pl.pallas_call(
        _tc_mid_body,
        grid=(NB,),
        in_specs=[
            pl.BlockSpec((NC, B, H), lambda i: (0, i, 0)),
            pl.BlockSpec((B, NW), lambda i: (i, 0)),
            pl.BlockSpec((B, H), lambda i: (i, 0)),
            pl.BlockSpec((B, 2), lambda i: (i, 0)),
            pl.BlockSpec((B, H), lambda i: (i, 0)),
            pl.BlockSpec((1, H), lambda i: (0, 0)),
            pl.BlockSpec((H, H), lambda i: (0, 0)),
            pl.BlockSpec((2, H), lambda i: (0, 0)),
        ],
        out_specs=[
            pl.BlockSpec((B, H), lambda i: (i, 0)),
            pl.BlockSpec((B, H), lambda i: (i, 0)),
            pl.BlockSpec((B, 2), lambda i: (i, 0)),
        ],
        out_shape=[
            jax.ShapeDtypeStruct((N, H), jnp.float32),
            jax.ShapeDtypeStruct((N, H), jnp.float32),
            jax.ShapeDtypeStruct((N, 2), jnp.float32),
        ],
    )(acc2, s32, hw, asad, hprev, bgr, Wg1, att1)


def _tc_final_body(acc2_ref, s32_ref, hw_ref, asad_ref, hprev_ref, bg_ref,
                   batch_ref, w2_ref, b2_ref, out_ref, pooled_ref):
    i = pl.program_id(0)
    h = _epilogue_h(acc2_ref, s32_ref, hw_ref, asad_ref, hprev_ref, bg_ref)
    gids = lax.broadcasted_iota(jnp.int32, (1, G), 1)
    onehot = (batch_ref[...] == gids).astype(jnp.float32)
    part = lax.dot_general(onehot, h, (((0,), (0,)), ((), ())),
                           preferred_element_type=jnp.float32)

    @pl.when(i == 0)
    def _():
        pooled_ref[...] = part

    @pl.when(i > 0)
    def _():
        pooled_ref[...] = pooled_ref[...] + part

    @pl.when(i == NB - 1)
    def _():
        out_ref[...] = jnp.dot(pooled_ref[...], w2_ref[...],
                               preferred_element_type=jnp.float32) + b2_ref[...]


def _tc_final(acc2, s32, hw, asad, hprev, bgr, batch_col, W2, b2r):
    return pl.pallas_call(
        _tc_final_body,
        grid=(NB,),
        in_specs=[
            pl.BlockSpec((NC, B, H), lambda i: (0, i, 0)),
            pl.BlockSpec((B, NW), lambda i: (i, 0)),
            pl.BlockSpec((B, H), lambda i: (i, 0)),
            pl.BlockSpec((B, 2), lambda i: (i, 0)),
            pl.BlockSpec((B, H), lambda i: (i, 0)),
            pl.BlockSpec((1, H), lambda i: (0, 0)),
            pl.BlockSpec((B, 1), lambda i: (i, 0)),
            pl.BlockSpec((H, O), lambda i: (0, 0)),
            pl.BlockSpec((1, O), lambda i: (0, 0)),
        ],
        out_specs=pl.BlockSpec((G, O), lambda i: (0, 0)),
        out_shape=jax.ShapeDtypeStruct((G, O), jnp.float32),
        scratch_shapes=[pltpu.VMEM((G, H), jnp.float32)],
    )(acc2, s32, hw, asad, hprev, bgr, batch_col, W2, b2r)


def _sc_edge_pass(hw, a_src, a_dst, srcp, dstp):
    mesh = plsc.VectorSubcoreMesh(core_axis_name="c", subcore_axis_name="s")

    @functools.partial(
        pl.kernel,
        mesh=mesh,
        compiler_params=pltpu.CompilerParams(needs_layout_passes=False),
        out_type=(
            jax.ShapeDtypeStruct((NC, N, H), jnp.float32),
            jax.ShapeDtypeStruct((NW, 1, N), jnp.float32),
        ),
        scratch_types=[
            pltpu.VMEM((N,), jnp.float32),
            pltpu.VMEM((N,), jnp.float32),
            pltpu.VMEM((N,), jnp.float32),
            pltpu.VMEM((CH, H), jnp.float32),
            pltpu.VMEM((CH,), jnp.int32),
            pltpu.VMEM((CH,), jnp.int32),
            pltpu.VMEM((CH,), jnp.float32),
            pltpu.VMEM_SHARED((N, H), jnp.float32),
            pltpu.SemaphoreType.DMA,
        ],
    )
    def k(hw_hbm, as_hbm, ad_hbm, src_hbm, dst_hbm, acc_out, s_out,
          asv, adv, s_loc, rows, sidx, didx, wbuf, acc_sh, gsem):
        cid = lax.axis_index("c")
        sid = lax.axis_index("s")
        wid = sid * NC + cid
        z16 = jnp.zeros((16,), jnp.float32)

        def zrow(j, _):
            for kk in range(H // 16):
                rows[j, pl.ds(kk * 16, 16)] = z16
            return 0
        lax.fori_loop(0, CH, zrow, 0)

        rbase = pl.multiple_of(sid * 624 + jnp.minimum(sid, 2) * 8, 8)
        for roff, nr in ZCHUNKS:
            pltpu.sync_copy(rows.at[pl.ds(0, nr)],
                            acc_sh.at[pl.ds(rbase + roff, nr)])

        @pl.when(sid < 2)
        def _():
            pltpu.sync_copy(rows.at[pl.ds(0, 8)],
                            acc_sh.at[pl.ds(rbase + 624, 8)])

        def zs(j, _):
            s_loc[pl.ds(j * 16, 16)] = z16
            return 0
        lax.fori_loop(0, N // 16, zs, 0)

        pltpu.sync_copy(as_hbm, asv)
        pltpu.sync_copy(ad_hbm, adv)

        plsc.subcore_barrier()

        lane = lax.iota(jnp.int32, 16)

        def chunk(i, _):
            base = wid * EPT + i * CH
            pltpu.sync_copy(src_hbm.at[pl.ds(base, CH)], sidx)
            pltpu.sync_copy(dst_hbm.at[pl.ds(base, CH)], didx)
            gd = pltpu.async_copy(hw_hbm.at[sidx], rows, gsem)
            for g in range(CH // 16):
                s16 = sidx[pl.ds(g * 16, 16)]
                d16 = didx[pl.ds(g * 16, 16)]
                a1 = plsc.load_gather(asv, [s16])
                a2 = plsc.load_gather(adv, [d16])
                w = jnp.exp(_leaky(a1 + a2))
                eoff = i * CH + g * 16 + lane
                w = jnp.where(eoff < EPT, w, 0.0)
                wbuf[pl.ds(g * 16, 16)] = w
                plsc.addupdate_scatter(s_loc, [d16], w)
            gd.wait()

            def scale(j, _):
                wv = plsc.load_gather(wbuf, [jnp.full((16,), j, jnp.int32)])
                for kk in range(H // 16):
                    rows[j, pl.ds(kk * 16, 16)] = \
                        rows[j, pl.ds(kk * 16, 16)] * wv
                return 0
            lax.fori_loop(0, CH, scale, 0)

            pltpu.sync_copy(rows, acc_sh.at[didx], add=True)
            return 0

        lax.fori_loop(0, NCHUNK, chunk, 0)

        plsc.subcore_barrier()

        for roff, nr in ZCHUNKS:
            pltpu.sync_copy(acc_sh.at[pl.ds(rbase + roff, nr)],
                            acc_out.at[cid, pl.ds(rbase + roff, nr)])

        @pl.when(sid < 2)
        def _():
            pltpu.sync_copy(acc_sh.at[pl.ds(rbase + 624, 8)],
                            acc_out.at[cid, pl.ds(rbase + 624, 8)])

        pltpu.sync_copy(s_loc, s_out.at[wid, 0])

    return k(hw, a_src, a_dst, srcp, dstp)


def kernel(x, edge_index, batch, W1, b1, Wg, att_src, att_dst, bg, W2, b2):
    pad = jnp.zeros((EPAD - E,), jnp.int32)
    srcp = jnp.concatenate([edge_index[0].astype(jnp.int32), pad])
    dstp = jnp.concatenate([edge_index[1].astype(jnp.int32), pad])
    batch_col = batch.astype(jnp.int32).reshape(N, 1)
    b1r = b1.reshape(1, H)
    b2r = b2.reshape(1, O)
    att0 = jnp.stack([att_src[0], att_dst[0]])
    att1 = jnp.stack([att_src[1], att_dst[1]])
    bg0 = bg[0].reshape(1, H)
    bg1 = bg[1].reshape(1, H)

    h0, hw0, asad0 = _tc_prep(x, W1, b1r, Wg[0], att0)
    acc0, s0 = _sc_edge_pass(hw0, asad0[:, 0], asad0[:, 1], srcp, dstp)
    h1, hw1, asad1 = _tc_mid(acc0, s0.reshape(NW, N).T, hw0, asad0, h0,
                             bg0, Wg[1], att1)
    acc1, s1 = _sc_edge_pass(hw1, asad1[:, 0], asad1[:, 1], srcp, dstp)
    return _tc_final(acc1, s1.reshape(NW, N).T, hw1, asad1, h1, bg1,
                     batch_col, W2, b2r)
